# baseline (speedup 1.0000x reference)
import functools

import jax
import jax.numpy as jnp
from jax import lax
from jax.experimental import pallas as pl
from jax.experimental.pallas import tpu as pltpu
from jax.experimental.pallas import tpu_sc as plsc

N = 10000
E = 320000
CW = 128
EP = 327680
EC = EP // CW
NPAD = 10112
NC = 2
NS = 16
SH = 624
TAIL = N - NS * SH
RB = 1000


def _sc_mesh():
    return plsc.VectorSubcoreMesh(core_axis_name="c", subcore_axis_name="s",
                                  num_cores=NC, num_subcores=NS)


def _make_deg():
    cpt = EC // (NC * NS)

    @functools.partial(
        pl.kernel,
        out_type=(jax.ShapeDtypeStruct((NPAD,), jnp.float32),
                  jax.ShapeDtypeStruct((NPAD,), jnp.float32)),
        mesh=_sc_mesh(),
        scratch_types=[
            pltpu.VMEM((cpt, CW), jnp.int32),
            pltpu.VMEM((CW,), jnp.float32),
            pltpu.VMEM_SHARED((NPAD,), jnp.float32),
        ],
    )
    def deg_kernel(dst2d, zeros_h, out0, out1, idst, onev, acc):
        c = lax.axis_index("c")
        s = lax.axis_index("s")

        @pl.when(s == 0)
        def _():
            pltpu.sync_copy(zeros_h, acc)

        for k in range(CW // 16):
            onev[pl.ds(k * 16, 16)] = jnp.ones((16,), jnp.float32)
        base = (c * NS + s) * cpt
        pltpu.sync_copy(dst2d.at[pl.ds(base, cpt), :], idst)
        plsc.subcore_barrier()

        def body(i, carry):
            pltpu.sync_copy(onev, acc.at[idst.at[i]], add=True)
            return carry

        lax.fori_loop(0, cpt, body, 0)
        plsc.subcore_barrier()

        @pl.when((s == 0) & (c == 0))
        def _():
            pltpu.sync_copy(acc, out0)

        @pl.when((s == 0) & (c == 1))
        def _():
            pltpu.sync_copy(acc, out1)

    return deg_kernel


def _make_agg(fh, feature_split):
    cpt = EC // NS if feature_split else EC // (NC * NS)

    gsz = 16

    @functools.partial(
        pl.kernel,
        out_type=(jax.ShapeDtypeStruct((N, fh), jnp.float32),
                  jax.ShapeDtypeStruct((N, fh), jnp.float32)),
        mesh=_sc_mesh(),
        scratch_types=[
            pltpu.VMEM((gsz, CW), jnp.int32),
            pltpu.VMEM((gsz, CW), jnp.int32),
            pltpu.VMEM((CW, fh), jnp.float32),
            pltpu.VMEM((CW, fh), jnp.float32),
            pltpu.VMEM_SHARED((NPAD, fh), jnp.float32),
            pltpu.SemaphoreType.DMA,
            pltpu.SemaphoreType.DMA,
            pltpu.SemaphoreType.DMA,
            pltpu.SemaphoreType.DMA,
        ],
    )
    def agg_kernel(g0, g1, src2d, dst2d, out0, out1,
                   isrc, idst, rows0, rows1, acc, sem0, sem1, ssem0, ssem1):
        c = lax.axis_index("c")
        s = lax.axis_index("s")
        if feature_split:
            base = s * cpt
        else:
            base = (c * NS + s) * cpt
        rbufs = (rows0, rows1)
        sems = (sem0, sem1)
        ssems = (ssem0, ssem1)

        def phase(g, out):
            pltpu.sync_copy(g.at[pl.ds(s * SH, SH), :],
                            acc.at[pl.ds(s * SH, SH), :])

            @pl.when(s == NS - 1)
            def _():
                pltpu.sync_copy(g.at[pl.ds(NS * SH, TAIL), :],
                                acc.at[pl.ds(NS * SH, TAIL), :])

            plsc.subcore_barrier()

            def group(gi, carry):
                pltpu.sync_copy(src2d.at[pl.ds(base + gi * gsz, gsz), :], isrc)
                pltpu.sync_copy(dst2d.at[pl.ds(base + gi * gsz, gsz), :], idst)
                dg = pltpu.async_copy(g.at[isrc.at[0]], rbufs[0], sems[0])
                sdesc = [None, None]
                for j in range(gsz):
                    dg.wait()
                    new_s = pltpu.async_copy(rbufs[j % 2], acc.at[idst.at[j]],
                                             ssems[j % 2], add=True)
                    if j + 1 < gsz:
                        if sdesc[(j + 1) % 2] is not None:
                            sdesc[(j + 1) % 2].wait()
                        dg = pltpu.async_copy(g.at[isrc.at[j + 1]],
                                              rbufs[(j + 1) % 2],
                                              sems[(j + 1) % 2])
                    sdesc[j % 2] = new_s
                sdesc[(gsz - 2) % 2].wait()
                sdesc[(gsz - 1) % 2].wait()
                return carry

            lax.fori_loop(0, cpt // gsz, group, 0)
            plsc.subcore_barrier()
            pltpu.sync_copy(acc.at[pl.ds(s * SH, SH), :],
                            out.at[pl.ds(s * SH, SH), :])

            @pl.when(s == NS - 1)
            def _():
                pltpu.sync_copy(acc.at[pl.ds(NS * SH, TAIL), :],
                                out.at[pl.ds(NS * SH, TAIL), :])

        @pl.when(c == 0)
        def _():
            phase(g0, out0)

        @pl.when(c == 1)
        def _():
            phase(g1, out1)

    return agg_kernel


def _rsqrt_deg(h_blk):
    return lax.rsqrt(h_blk[:, 0] + h_blk[:, 1] + 1.0)


def _tc1(x, w1, hist_t):
    def body(x_ref, w_ref, h_ref, o0_ref, o1_ref):
        r = _rsqrt_deg(h_ref[...])
        g = jnp.dot(x_ref[...], w_ref[...],
                    preferred_element_type=jnp.float32,
                    precision=lax.Precision.HIGHEST) * r[:, None]
        o0_ref[...] = g[:, :128]
        o1_ref[...] = g[:, 128:]

    return pl.pallas_call(
        body,
        grid=(N // RB,),
        in_specs=[
            pl.BlockSpec((RB, 128), lambda i: (i, 0)),
            pl.BlockSpec((128, 256), lambda i: (0, 0)),
            pl.BlockSpec((RB, 2), lambda i: (i, 0)),
        ],
        out_specs=(pl.BlockSpec((RB, 128), lambda i: (i, 0)),
                   pl.BlockSpec((RB, 128), lambda i: (i, 0))),
        out_shape=(jax.ShapeDtypeStruct((N, 128), jnp.float32),
                   jax.ShapeDtypeStruct((N, 128), jnp.float32)),
    )(x, w1, hist_t)


def _tc_mid(s0, s1, hist_t, b, w, out_w, split_out):
    def body(s0_ref, s1_ref, h_ref, b_ref, w_ref, *outs):
        r = _rsqrt_deg(h_ref[...])[:, None]
        a0 = jnp.maximum(r * s0_ref[...] + b_ref[0, :128], 0.0)
        a1 = jnp.maximum(r * s1_ref[...] + b_ref[0, 128:], 0.0)
        g = (jnp.dot(a0, w_ref[:128, :], preferred_element_type=jnp.float32,
                    precision=lax.Precision.HIGHEST)
             + jnp.dot(a1, w_ref[128:, :], preferred_element_type=jnp.float32,
                    precision=lax.Precision.HIGHEST)
             ) * r
        if split_out:
            outs[0][...] = g[:, : out_w // 2]
            outs[1][...] = g[:, out_w // 2:]
        else:
            outs[0][...] = jnp.concatenate(
                [g, jnp.zeros((g.shape[0], 128 - out_w), jnp.float32)], axis=1)

    if split_out:
        out_specs = (pl.BlockSpec((RB, out_w // 2), lambda i: (i, 0)),
                     pl.BlockSpec((RB, out_w // 2), lambda i: (i, 0)))
        out_shape = (jax.ShapeDtypeStruct((N, out_w // 2), jnp.float32),
                     jax.ShapeDtypeStruct((N, out_w // 2), jnp.float32))
    else:
        out_specs = pl.BlockSpec((RB, 128), lambda i: (i, 0))
        out_shape = jax.ShapeDtypeStruct((N, 128), jnp.float32)

    return pl.pallas_call(
        body,
        grid=(N // RB,),
        in_specs=[
            pl.BlockSpec((RB, 128), lambda i: (i, 0)),
            pl.BlockSpec((RB, 128), lambda i: (i, 0)),
            pl.BlockSpec((RB, 2), lambda i: (i, 0)),
            pl.BlockSpec((1, 256), lambda i: (0, 0)),
            pl.BlockSpec((256, out_w), lambda i: (0, 0)),
        ],
        out_specs=out_specs,
        out_shape=out_shape,
    )(s0, s1, hist_t, b, w)


def _tc4(s0, s1, g3, hist_t, b3, wc, bc):
    def body(s0_ref, s1_ref, g_ref, h_ref, b_ref, wc_ref, bc_ref, o_ref):
        r = _rsqrt_deg(h_ref[...])[:, None]
        u = r * (s0_ref[...] + s1_ref[...] - g_ref[...])[:, :64]
        hh = jnp.maximum(u + b_ref[0], 0.0)
        o_ref[...] = (jnp.dot(hh, wc_ref[...],
                              preferred_element_type=jnp.float32,
                    precision=lax.Precision.HIGHEST) + bc_ref[0])

    nclass = wc.shape[1]
    return pl.pallas_call(
        body,
        grid=(N // RB,),
        in_specs=[
            pl.BlockSpec((RB, 128), lambda i: (i, 0)),
            pl.BlockSpec((RB, 128), lambda i: (i, 0)),
            pl.BlockSpec((RB, 128), lambda i: (i, 0)),
            pl.BlockSpec((RB, 2), lambda i: (i, 0)),
            pl.BlockSpec((1, 64), lambda i: (0, 0)),
            pl.BlockSpec((64, nclass), lambda i: (0, 0)),
            pl.BlockSpec((1, nclass), lambda i: (0, 0)),
        ],
        out_specs=pl.BlockSpec((RB, nclass), lambda i: (i, 0)),
        out_shape=jax.ShapeDtypeStruct((N, nclass), jnp.float32),
    )(s0, s1, g3, hist_t, b3, wc, bc)


_deg = _make_deg()
_agg256 = _make_agg(128, True)
_agg64 = _make_agg(128, False)


def kernel(x, edge_index, W1, b1, W2, b2, W3, b3, Wc, bc):
    src = edge_index[0]
    dst = edge_index[1]
    pad = jnp.arange(EP - E, dtype=jnp.int32) % 16
    src_p = jnp.concatenate([src, pad]).reshape(EC, CW)
    dst_p = jnp.concatenate([dst, pad + N]).reshape(EC, CW)
    zeros_h = jnp.zeros((NPAD,), jnp.float32)

    h0, h1 = _deg(dst_p, zeros_h)
    hist_t = jnp.stack([h0[:N], h1[:N]], axis=1)

    g1a, g1b = _tc1(x, W1, hist_t)
    s1a, s1b = _agg256(g1a, g1b, src_p, dst_p)
    g2a, g2b = _tc_mid(s1a, s1b, hist_t, b1.reshape(1, 256), W2, 256, True)
    s2a, s2b = _agg256(g2a, g2b, src_p, dst_p)
    g3 = _tc_mid(s2a, s2b, hist_t, b2.reshape(1, 256), W3, 64, False)
    s3a, s3b = _agg64(g3, g3, src_p, dst_p)
    return _tc4(s3a, s3b, g3, hist_t, b3.reshape(1, 64), Wc, bc.reshape(1, 7))


# speedup vs baseline: 1.1397x; 1.1397x over previous
import functools

import jax
import jax.numpy as jnp
from jax import lax
from jax.experimental import pallas as pl
from jax.experimental.pallas import tpu as pltpu
from jax.experimental.pallas import tpu_sc as plsc

N = 10000
E = 320000
CW = 128
EP = 327680
EC = EP // CW
NPAD = 10112
NC = 2
NS = 16
SH = 624
TAIL = N - NS * SH
RB = 1000


def _sc_mesh():
    return plsc.VectorSubcoreMesh(core_axis_name="c", subcore_axis_name="s",
                                  num_cores=NC, num_subcores=NS)


def _make_deg():
    cpt = EC // (NC * NS)

    @functools.partial(
        pl.kernel,
        out_type=(jax.ShapeDtypeStruct((NPAD,), jnp.float32),
                  jax.ShapeDtypeStruct((NPAD,), jnp.float32)),
        mesh=_sc_mesh(),
        scratch_types=[
            pltpu.VMEM((cpt, CW), jnp.int32),
            pltpu.VMEM((CW,), jnp.float32),
            pltpu.VMEM_SHARED((NPAD,), jnp.float32),
        ],
    )
    def deg_kernel(dst2d, zeros_h, out0, out1, idst, onev, acc):
        c = lax.axis_index("c")
        s = lax.axis_index("s")

        @pl.when(s == 0)
        def _():
            pltpu.sync_copy(zeros_h, acc)

        for k in range(CW // 16):
            onev[pl.ds(k * 16, 16)] = jnp.ones((16,), jnp.float32)
        base = (c * NS + s) * cpt
        pltpu.sync_copy(dst2d.at[pl.ds(base, cpt), :], idst)
        plsc.subcore_barrier()

        def body(i, carry):
            pltpu.sync_copy(onev, acc.at[idst.at[i]], add=True)
            return carry

        lax.fori_loop(0, cpt, body, 0)
        plsc.subcore_barrier()

        @pl.when((s == 0) & (c == 0))
        def _():
            pltpu.sync_copy(acc, out0)

        @pl.when((s == 0) & (c == 1))
        def _():
            pltpu.sync_copy(acc, out1)

    return deg_kernel


def _make_agg(fh, feature_split):
    cpt = EC // NS if feature_split else EC // (NC * NS)

    gsz = 16

    @functools.partial(
        pl.kernel,
        out_type=(jax.ShapeDtypeStruct((N, fh), jnp.float32),
                  jax.ShapeDtypeStruct((N, fh), jnp.float32)),
        mesh=_sc_mesh(),
        scratch_types=[
            pltpu.VMEM((gsz, CW), jnp.int32),
            pltpu.VMEM((gsz, CW), jnp.int32),
            pltpu.VMEM((CW, fh), jnp.float32),
            pltpu.VMEM((CW, fh), jnp.float32),
            pltpu.VMEM_SHARED((NPAD, fh), jnp.float32),
            pltpu.SemaphoreType.DMA,
            pltpu.SemaphoreType.DMA,
            pltpu.SemaphoreType.DMA,
            pltpu.SemaphoreType.DMA,
        ],
    )
    def agg_kernel(g0, g1, src2d, dst2d, out0, out1,
                   isrc, idst, rows0, rows1, acc, sem0, sem1, ssem0, ssem1):
        c = lax.axis_index("c")
        s = lax.axis_index("s")
        if feature_split:
            base = s * cpt
        else:
            base = (c * NS + s) * cpt
        rbufs = (rows0, rows1)
        sems = (sem0, sem1)
        ssems = (ssem0, ssem1)

        def phase(g, out):
            pltpu.sync_copy(g.at[pl.ds(s * SH, SH), :],
                            acc.at[pl.ds(s * SH, SH), :])

            @pl.when(s == NS - 1)
            def _():
                pltpu.sync_copy(g.at[pl.ds(NS * SH, TAIL), :],
                                acc.at[pl.ds(NS * SH, TAIL), :])

            plsc.subcore_barrier()

            def group(gi, carry):
                pltpu.sync_copy(src2d.at[pl.ds(base + gi * gsz, gsz), :], isrc)
                pltpu.sync_copy(dst2d.at[pl.ds(base + gi * gsz, gsz), :], idst)
                desc = pltpu.async_copy(g.at[isrc.at[0]], rbufs[0], sems[0])
                for j in range(gsz):
                    if j + 1 < gsz:
                        nxt = pltpu.async_copy(g.at[isrc.at[j + 1]],
                                               rbufs[(j + 1) % 2],
                                               sems[(j + 1) % 2])
                    desc.wait()
                    pltpu.sync_copy(rbufs[j % 2], acc.at[idst.at[j]], add=True)
                    if j + 1 < gsz:
                        desc = nxt
                return carry

            lax.fori_loop(0, cpt // gsz, group, 0)
            plsc.subcore_barrier()
            pltpu.sync_copy(acc.at[pl.ds(s * SH, SH), :],
                            out.at[pl.ds(s * SH, SH), :])

            @pl.when(s == NS - 1)
            def _():
                pltpu.sync_copy(acc.at[pl.ds(NS * SH, TAIL), :],
                                out.at[pl.ds(NS * SH, TAIL), :])

        @pl.when(c == 0)
        def _():
            phase(g0, out0)

        @pl.when(c == 1)
        def _():
            phase(g1, out1)

    return agg_kernel


def _rsqrt_deg(h_blk):
    return lax.rsqrt(h_blk[:, 0] + h_blk[:, 1] + 1.0)


def _tc1(x, w1, hist_t):
    def body(x_ref, w_ref, h_ref, o0_ref, o1_ref):
        r = _rsqrt_deg(h_ref[...])
        g = jnp.dot(x_ref[...], w_ref[...],
                    preferred_element_type=jnp.float32,
                    precision=lax.Precision.HIGHEST) * r[:, None]
        o0_ref[...] = g[:, :128]
        o1_ref[...] = g[:, 128:]

    return pl.pallas_call(
        body,
        grid=(N // RB,),
        in_specs=[
            pl.BlockSpec((RB, 128), lambda i: (i, 0)),
            pl.BlockSpec((128, 256), lambda i: (0, 0)),
            pl.BlockSpec((RB, 2), lambda i: (i, 0)),
        ],
        out_specs=(pl.BlockSpec((RB, 128), lambda i: (i, 0)),
                   pl.BlockSpec((RB, 128), lambda i: (i, 0))),
        out_shape=(jax.ShapeDtypeStruct((N, 128), jnp.float32),
                   jax.ShapeDtypeStruct((N, 128), jnp.float32)),
    )(x, w1, hist_t)


def _tc_mid(s0, s1, hist_t, b, w, out_w, split_out):
    def body(s0_ref, s1_ref, h_ref, b_ref, w_ref, *outs):
        r = _rsqrt_deg(h_ref[...])[:, None]
        a0 = jnp.maximum(r * s0_ref[...] + b_ref[0, :128], 0.0)
        a1 = jnp.maximum(r * s1_ref[...] + b_ref[0, 128:], 0.0)
        g = (jnp.dot(a0, w_ref[:128, :], preferred_element_type=jnp.float32,
                    precision=lax.Precision.HIGHEST)
             + jnp.dot(a1, w_ref[128:, :], preferred_element_type=jnp.float32,
                    precision=lax.Precision.HIGHEST)
             ) * r
        if split_out:
            outs[0][...] = g[:, : out_w // 2]
            outs[1][...] = g[:, out_w // 2:]
        else:
            outs[0][...] = jnp.concatenate(
                [g, jnp.zeros((g.shape[0], 128 - out_w), jnp.float32)], axis=1)

    if split_out:
        out_specs = (pl.BlockSpec((RB, out_w // 2), lambda i: (i, 0)),
                     pl.BlockSpec((RB, out_w // 2), lambda i: (i, 0)))
        out_shape = (jax.ShapeDtypeStruct((N, out_w // 2), jnp.float32),
                     jax.ShapeDtypeStruct((N, out_w // 2), jnp.float32))
    else:
        out_specs = pl.BlockSpec((RB, 128), lambda i: (i, 0))
        out_shape = jax.ShapeDtypeStruct((N, 128), jnp.float32)

    return pl.pallas_call(
        body,
        grid=(N // RB,),
        in_specs=[
            pl.BlockSpec((RB, 128), lambda i: (i, 0)),
            pl.BlockSpec((RB, 128), lambda i: (i, 0)),
            pl.BlockSpec((RB, 2), lambda i: (i, 0)),
            pl.BlockSpec((1, 256), lambda i: (0, 0)),
            pl.BlockSpec((256, out_w), lambda i: (0, 0)),
        ],
        out_specs=out_specs,
        out_shape=out_shape,
    )(s0, s1, hist_t, b, w)


def _tc4(s0, s1, g3, hist_t, b3, wc, bc):
    def body(s0_ref, s1_ref, g_ref, h_ref, b_ref, wc_ref, bc_ref, o_ref):
        r = _rsqrt_deg(h_ref[...])[:, None]
        u = r * (s0_ref[...] + s1_ref[...] - g_ref[...])[:, :64]
        hh = jnp.maximum(u + b_ref[0], 0.0)
        o_ref[...] = (jnp.dot(hh, wc_ref[...],
                              preferred_element_type=jnp.float32,
                    precision=lax.Precision.HIGHEST) + bc_ref[0])

    nclass = wc.shape[1]
    return pl.pallas_call(
        body,
        grid=(N // RB,),
        in_specs=[
            pl.BlockSpec((RB, 128), lambda i: (i, 0)),
            pl.BlockSpec((RB, 128), lambda i: (i, 0)),
            pl.BlockSpec((RB, 128), lambda i: (i, 0)),
            pl.BlockSpec((RB, 2), lambda i: (i, 0)),
            pl.BlockSpec((1, 64), lambda i: (0, 0)),
            pl.BlockSpec((64, nclass), lambda i: (0, 0)),
            pl.BlockSpec((1, nclass), lambda i: (0, 0)),
        ],
        out_specs=pl.BlockSpec((RB, nclass), lambda i: (i, 0)),
        out_shape=jax.ShapeDtypeStruct((N, nclass), jnp.float32),
    )(s0, s1, g3, hist_t, b3, wc, bc)


_deg = _make_deg()
_agg256 = _make_agg(128, True)
_agg64 = _make_agg(128, False)


def kernel(x, edge_index, W1, b1, W2, b2, W3, b3, Wc, bc):
    src = edge_index[0]
    dst = edge_index[1]
    pad = jnp.arange(EP - E, dtype=jnp.int32) % 16
    src_p = jnp.concatenate([src, pad]).reshape(EC, CW)
    dst_p = jnp.concatenate([dst, pad + N]).reshape(EC, CW)
    zeros_h = jnp.zeros((NPAD,), jnp.float32)

    h0, h1 = _deg(dst_p, zeros_h)
    hist_t = jnp.stack([h0[:N], h1[:N]], axis=1)

    g1a, g1b = _tc1(x, W1, hist_t)
    s1a, s1b = _agg256(g1a, g1b, src_p, dst_p)
    g2a, g2b = _tc_mid(s1a, s1b, hist_t, b1.reshape(1, 256), W2, 256, True)
    s2a, s2b = _agg256(g2a, g2b, src_p, dst_p)
    g3 = _tc_mid(s2a, s2b, hist_t, b2.reshape(1, 256), W3, 64, False)
    s3a, s3b = _agg64(g3, g3, src_p, dst_p)
    return _tc4(s3a, s3b, g3, hist_t, b3.reshape(1, 64), Wc, bc.reshape(1, 7))


# speedup vs baseline: 1.1498x; 1.0089x over previous
import functools

import jax
import jax.numpy as jnp
from jax import lax
from jax.experimental import pallas as pl
from jax.experimental.pallas import tpu as pltpu
from jax.experimental.pallas import tpu_sc as plsc

N = 10000
E = 320000
CW = 128
EP = 327680
EC = EP // CW
NPAD = 10112
NC = 2
NS = 16
SH = 624
TAIL = N - NS * SH
RB = 1000


def _sc_mesh():
    return plsc.VectorSubcoreMesh(core_axis_name="c", subcore_axis_name="s",
                                  num_cores=NC, num_subcores=NS)


def _make_deg():
    cpt = EC // (NC * NS)

    @functools.partial(
        pl.kernel,
        out_type=(jax.ShapeDtypeStruct((NPAD,), jnp.float32),
                  jax.ShapeDtypeStruct((NPAD,), jnp.float32)),
        mesh=_sc_mesh(),
        scratch_types=[
            pltpu.VMEM((cpt, CW), jnp.int32),
            pltpu.VMEM((CW,), jnp.float32),
            pltpu.VMEM_SHARED((NPAD,), jnp.float32),
        ],
    )
    def deg_kernel(dst2d, zeros_h, out0, out1, idst, onev, acc):
        c = lax.axis_index("c")
        s = lax.axis_index("s")

        @pl.when(s == 0)
        def _():
            pltpu.sync_copy(zeros_h, acc)

        for k in range(CW // 16):
            onev[pl.ds(k * 16, 16)] = jnp.ones((16,), jnp.float32)
        base = (c * NS + s) * cpt
        pltpu.sync_copy(dst2d.at[pl.ds(base, cpt), :], idst)
        plsc.subcore_barrier()

        def body(i, carry):
            pltpu.sync_copy(onev, acc.at[idst.at[i]], add=True)
            return carry

        lax.fori_loop(0, cpt, body, 0)
        plsc.subcore_barrier()

        @pl.when((s == 0) & (c == 0))
        def _():
            pltpu.sync_copy(acc, out0)

        @pl.when((s == 0) & (c == 1))
        def _():
            pltpu.sync_copy(acc, out1)

    return deg_kernel


def _make_agg(fh, feature_split, rw=None):
    cpt = EC // NS if feature_split else EC // (NC * NS)
    rw = fh if rw is None else rw

    gsz = 16

    @functools.partial(
        pl.kernel,
        out_type=(jax.ShapeDtypeStruct((N, rw), jnp.float32),
                  jax.ShapeDtypeStruct((N, rw), jnp.float32)),
        mesh=_sc_mesh(),
        scratch_types=[
            pltpu.VMEM((gsz, CW), jnp.int32),
            pltpu.VMEM((gsz, CW), jnp.int32),
            pltpu.VMEM((CW, fh), jnp.float32),
            pltpu.VMEM((CW, fh), jnp.float32),
            pltpu.VMEM_SHARED((NPAD, rw), jnp.float32),
            pltpu.SemaphoreType.DMA,
            pltpu.SemaphoreType.DMA,
        ],
    )
    def agg_kernel(g0, g1, gc, src2d, dst2d, out0, out1,
                   isrc, idst, rows0, rows1, acc, sem0, sem1):
        c = lax.axis_index("c")
        s = lax.axis_index("s")
        if feature_split:
            base = s * cpt
        else:
            base = (c * NS + s) * cpt
        rbufs = (rows0, rows1)
        sems = (sem0, sem1)

        def sbuf(rb):
            return rb if rw == fh else rb.at[:, pl.ds(0, rw)]

        def phase(g, out):
            gi_t = g if rw == fh else gc
            pltpu.sync_copy(gi_t.at[pl.ds(s * SH, SH), :],
                            acc.at[pl.ds(s * SH, SH), :])

            @pl.when(s == NS - 1)
            def _():
                pltpu.sync_copy(gi_t.at[pl.ds(NS * SH, TAIL), :],
                                acc.at[pl.ds(NS * SH, TAIL), :])

            plsc.subcore_barrier()

            def group(gi, carry):
                pltpu.sync_copy(src2d.at[pl.ds(base + gi * gsz, gsz), :], isrc)
                pltpu.sync_copy(dst2d.at[pl.ds(base + gi * gsz, gsz), :], idst)
                desc = pltpu.async_copy(g.at[isrc.at[0]], rbufs[0], sems[0])
                for j in range(gsz):
                    if j + 1 < gsz:
                        nxt = pltpu.async_copy(g.at[isrc.at[j + 1]],
                                               rbufs[(j + 1) % 2],
                                               sems[(j + 1) % 2])
                    desc.wait()
                    pltpu.sync_copy(sbuf(rbufs[j % 2]), acc.at[idst.at[j]],
                                    add=True)
                    if j + 1 < gsz:
                        desc = nxt
                return carry

            lax.fori_loop(0, cpt // gsz, group, 0)
            plsc.subcore_barrier()
            pltpu.sync_copy(acc.at[pl.ds(s * SH, SH), :],
                            out.at[pl.ds(s * SH, SH), :])

            @pl.when(s == NS - 1)
            def _():
                pltpu.sync_copy(acc.at[pl.ds(NS * SH, TAIL), :],
                                out.at[pl.ds(NS * SH, TAIL), :])

        @pl.when(c == 0)
        def _():
            phase(g0, out0)

        @pl.when(c == 1)
        def _():
            phase(g1, out1)

    return agg_kernel


def _rsqrt_deg(h_blk):
    return lax.rsqrt(h_blk[:, 0] + h_blk[:, 1] + 1.0)


def _tc1(x, w1, hist_t):
    def body(x_ref, w_ref, h_ref, o0_ref, o1_ref):
        r = _rsqrt_deg(h_ref[...])
        g = jnp.dot(x_ref[...], w_ref[...],
                    preferred_element_type=jnp.float32,
                    precision=lax.Precision.HIGHEST) * r[:, None]
        o0_ref[...] = g[:, :128]
        o1_ref[...] = g[:, 128:]

    return pl.pallas_call(
        body,
        grid=(N // RB,),
        in_specs=[
            pl.BlockSpec((RB, 128), lambda i: (i, 0)),
            pl.BlockSpec((128, 256), lambda i: (0, 0)),
            pl.BlockSpec((RB, 2), lambda i: (i, 0)),
        ],
        out_specs=(pl.BlockSpec((RB, 128), lambda i: (i, 0)),
                   pl.BlockSpec((RB, 128), lambda i: (i, 0))),
        out_shape=(jax.ShapeDtypeStruct((N, 128), jnp.float32),
                   jax.ShapeDtypeStruct((N, 128), jnp.float32)),
    )(x, w1, hist_t)


def _tc_mid(s0, s1, hist_t, b, w, out_w, split_out):
    def body(s0_ref, s1_ref, h_ref, b_ref, w_ref, *outs):
        r = _rsqrt_deg(h_ref[...])[:, None]
        a0 = jnp.maximum(r * s0_ref[...] + b_ref[0, :128], 0.0)
        a1 = jnp.maximum(r * s1_ref[...] + b_ref[0, 128:], 0.0)
        g = (jnp.dot(a0, w_ref[:128, :], preferred_element_type=jnp.float32,
                    precision=lax.Precision.HIGHEST)
             + jnp.dot(a1, w_ref[128:, :], preferred_element_type=jnp.float32,
                    precision=lax.Precision.HIGHEST)
             ) * r
        if split_out:
            outs[0][...] = g[:, : out_w // 2]
            outs[1][...] = g[:, out_w // 2:]
        else:
            outs[0][...] = jnp.concatenate(
                [g, jnp.zeros((g.shape[0], 128 - out_w), jnp.float32)], axis=1)
            outs[1][...] = g

    if split_out:
        out_specs = (pl.BlockSpec((RB, out_w // 2), lambda i: (i, 0)),
                     pl.BlockSpec((RB, out_w // 2), lambda i: (i, 0)))
        out_shape = (jax.ShapeDtypeStruct((N, out_w // 2), jnp.float32),
                     jax.ShapeDtypeStruct((N, out_w // 2), jnp.float32))
    else:
        out_specs = (pl.BlockSpec((RB, 128), lambda i: (i, 0)),
                     pl.BlockSpec((RB, out_w), lambda i: (i, 0)))
        out_shape = (jax.ShapeDtypeStruct((N, 128), jnp.float32),
                     jax.ShapeDtypeStruct((N, out_w), jnp.float32))

    return pl.pallas_call(
        body,
        grid=(N // RB,),
        in_specs=[
            pl.BlockSpec((RB, 128), lambda i: (i, 0)),
            pl.BlockSpec((RB, 128), lambda i: (i, 0)),
            pl.BlockSpec((RB, 2), lambda i: (i, 0)),
            pl.BlockSpec((1, 256), lambda i: (0, 0)),
            pl.BlockSpec((256, out_w), lambda i: (0, 0)),
        ],
        out_specs=out_specs,
        out_shape=out_shape,
    )(s0, s1, hist_t, b, w)


def _tc4(s0, s1, g3, hist_t, b3, wc, bc):
    def body(s0_ref, s1_ref, g_ref, h_ref, b_ref, wc_ref, bc_ref, o_ref):
        r = _rsqrt_deg(h_ref[...])[:, None]
        u = r * ((s0_ref[...] + s1_ref[...])[:, :64] - g_ref[...])
        hh = jnp.maximum(u + b_ref[0], 0.0)
        o_ref[...] = (jnp.dot(hh, wc_ref[...],
                              preferred_element_type=jnp.float32,
                    precision=lax.Precision.HIGHEST) + bc_ref[0])

    nclass = wc.shape[1]
    return pl.pallas_call(
        body,
        grid=(N // RB,),
        in_specs=[
            pl.BlockSpec((RB, 128), lambda i: (i, 0)),
            pl.BlockSpec((RB, 128), lambda i: (i, 0)),
            pl.BlockSpec((RB, 64), lambda i: (i, 0)),
            pl.BlockSpec((RB, 2), lambda i: (i, 0)),
            pl.BlockSpec((1, 64), lambda i: (0, 0)),
            pl.BlockSpec((64, nclass), lambda i: (0, 0)),
            pl.BlockSpec((1, nclass), lambda i: (0, 0)),
        ],
        out_specs=pl.BlockSpec((RB, nclass), lambda i: (i, 0)),
        out_shape=jax.ShapeDtypeStruct((N, nclass), jnp.float32),
    )(s0, s1, g3, hist_t, b3, wc, bc)


_deg = _make_deg()
_agg256 = _make_agg(128, True)
_agg64 = _make_agg(128, False)


def kernel(x, edge_index, W1, b1, W2, b2, W3, b3, Wc, bc):
    src = edge_index[0]
    dst = edge_index[1]
    pad = jnp.arange(EP - E, dtype=jnp.int32) % 16
    src_p = jnp.concatenate([src, pad]).reshape(EC, CW)
    dst_p = jnp.concatenate([dst, pad + N]).reshape(EC, CW)
    zeros_h = jnp.zeros((NPAD,), jnp.float32)

    h0, h1 = _deg(dst_p, zeros_h)
    hist_t = jnp.stack([h0[:N], h1[:N]], axis=1)

    g1a, g1b = _tc1(x, W1, hist_t)
    s1a, s1b = _agg256(g1a, g1b, g1a, src_p, dst_p)
    g2a, g2b = _tc_mid(s1a, s1b, hist_t, b1.reshape(1, 256), W2, 256, True)
    s2a, s2b = _agg256(g2a, g2b, g2a, src_p, dst_p)
    g3p, g3c = _tc_mid(s2a, s2b, hist_t, b2.reshape(1, 256), W3, 64, False)
    s3a, s3b = _agg64(g3p, g3p, g3c, src_p, dst_p)
    return _tc4(s3a, s3b, g3c, hist_t, b3.reshape(1, 64), Wc, bc.reshape(1, 7))
